# baseline (speedup 1.0000x reference)
import functools

import jax
import jax.numpy as jnp
from jax import lax
from jax.experimental import pallas as pl
from jax.experimental.pallas import tpu as pltpu
from jax.experimental.pallas import tpu_sc as plsc

VOCAB = 4096
DIM = 4096
ROWS = 16384
NW = 32
RPW = ROWS // NW
CH = 4
NBUF = 5
NCH = RPW // CH
ECH = 128
NECH = RPW // ECH
LSE_BLOCK = 512


def _lse_body(t_ref, o_ref):
    x = t_ref[...]
    m = jnp.max(x, axis=1, keepdims=True)
    s = jnp.sum(jnp.exp(x - m), axis=1, keepdims=True)
    o_ref[...] = m + jnp.log(s)


def _row_lse(table):
    return pl.pallas_call(
        _lse_body,
        grid=(VOCAB // LSE_BLOCK,),
        in_specs=[pl.BlockSpec((LSE_BLOCK, DIM), lambda i: (i, 0))],
        out_specs=pl.BlockSpec((LSE_BLOCK, 1), lambda i: (i, 0)),
        out_shape=jax.ShapeDtypeStruct((VOCAB, 1), jnp.float32),
    )(table)


def _sc_row_gather(table, idx2, t2):
    mesh = plsc.VectorSubcoreMesh(core_axis_name="c", subcore_axis_name="s")

    @functools.partial(
        pl.kernel,
        out_type=(jax.ShapeDtypeStruct((ROWS, DIM), jnp.float32),
                  jax.ShapeDtypeStruct((NW, 16), jnp.float32)),
        mesh=mesh,
        scratch_types=[
            pltpu.VMEM((NCH, 16), jnp.int32),
            pltpu.VMEM((NCH, 16), jnp.int32),
            pltpu.VMEM((16,), jnp.float32),
            pltpu.VMEM((CH, DIM), jnp.float32),
            pltpu.VMEM((CH, DIM), jnp.float32),
            pltpu.VMEM((CH, DIM), jnp.float32),
            pltpu.VMEM((CH, DIM), jnp.float32),
            pltpu.VMEM((CH, DIM), jnp.float32),
            pltpu.SemaphoreType.DMA,
            pltpu.SemaphoreType.DMA,
            pltpu.SemaphoreType.DMA,
            pltpu.SemaphoreType.DMA,
            pltpu.SemaphoreType.DMA,
            pltpu.SemaphoreType.DMA,
            pltpu.SemaphoreType.DMA,
            pltpu.SemaphoreType.DMA,
            pltpu.SemaphoreType.DMA,
            pltpu.SemaphoreType.DMA,
        ],
    )
    def k(table_hbm, idx2_hbm, t2_hbm, out_hbm, tgt_hbm,
          idx2_v, t2_v, acc_v, b0, b1, b2, b3, b4,
          g0, g1, g2, g3, g4, o0, o1, o2, o3, o4):
        bufs = (b0, b1, b2, b3, b4)
        gsems = (g0, g1, g2, g3, g4)
        osems = (o0, o1, o2, o3, o4)
        wid = lax.axis_index("s") * 2 + lax.axis_index("c")
        base = wid * RPW
        cbase = wid * NCH
        pltpu.sync_copy(idx2_hbm.at[pl.ds(cbase, NCH)], idx2_v)
        pltpu.sync_copy(t2_hbm.at[pl.ds(cbase, NCH)], t2_v)
        acc_v[...] = jnp.zeros((16,), jnp.float32)
        lanes = lax.iota(jnp.int32, 16)
        pltpu.async_copy(table_hbm.at[idx2_v.at[0, pl.ds(0, CH)]],
                         bufs[0], gsems[0])
        pltpu.async_copy(table_hbm.at[idx2_v.at[1, pl.ds(0, CH)]],
                         bufs[1], gsems[1])

        @pl.loop(0, NCH + 7, step=NBUF)
        def _(gbase):
            for j in range(NBUF):
                g = gbase + j
                b = j
                nb = (j + 2) % NBUF

                @pl.when(jnp.logical_and(g >= 3, g < NCH + 3))
                def _():
                    pltpu.make_async_copy(
                        bufs[nb], out_hbm.at[pl.ds(base, CH)],
                        osems[nb]).wait()

                @pl.when(g + 2 < NCH)
                def _():
                    pltpu.async_copy(
                        table_hbm.at[idx2_v.at[g + 2, pl.ds(0, CH)]],
                        bufs[nb], gsems[nb])

                @pl.when(g < NCH)
                def _():
                    pltpu.make_async_copy(
                        table_hbm.at[idx2_v.at[0, pl.ds(0, CH)]],
                        bufs[b], gsems[b]).wait()
                    t16 = t2_v[g]
                    for r in range(CH):
                        t_r = t16[r]
                        a_r = pl.multiple_of((t_r // 16) * 16, 16)
                        v16 = bufs[b][r, pl.ds(a_r, 16)]
                        sel = jnp.where(lanes == t_r - a_r, v16,
                                        jnp.zeros((16,), jnp.float32))
                        acc_v[...] = acc_v[...] + sel
                    pltpu.async_copy(bufs[b],
                                     out_hbm.at[pl.ds(base + g * CH, CH)],
                                     osems[b])

        pltpu.sync_copy(acc_v, tgt_hbm.at[wid])

    return k(table, idx2, t2)


def _sc_lse_gather(lse_flat, idx_flat, tgt_acc):
    mesh = plsc.VectorSubcoreMesh(core_axis_name="c", subcore_axis_name="s")

    @functools.partial(
        pl.kernel,
        out_type=jax.ShapeDtypeStruct((ROWS,), jnp.float32),
        mesh=mesh,
        scratch_types=[
            pltpu.VMEM((RPW,), jnp.int32),
            pltpu.VMEM((RPW,), jnp.float32),
            pltpu.SemaphoreType.DMA,
        ],
    )
    def k(lse_hbm, idx_hbm, acc_hbm, lseg_hbm, idx_v, lseg_v, sem):
        wid = lax.axis_index("s") * 2 + lax.axis_index("c")
        base = wid * RPW
        pltpu.sync_copy(idx_hbm.at[pl.ds(base, RPW)], idx_v)

        @pl.loop(0, NECH)
        def _(e):
            pltpu.async_copy(
                lse_hbm.at[idx_v.at[pl.ds(e * ECH, ECH)]],
                lseg_v.at[pl.ds(e * ECH, ECH)], sem).wait()

        pltpu.sync_copy(lseg_v, lseg_hbm.at[pl.ds(base, RPW)])

    return k(lse_flat, idx_flat, tgt_acc)


def _loss_body(a_ref, b_ref, o_ref):
    sa = jnp.sum(a_ref[...], keepdims=True)
    sb = jnp.sum(b_ref[...], keepdims=True)
    o_ref[...] = (sa - sb) * (1.0 / float(ROWS))


def _loss(lse_g, tgt_acc):
    out = pl.pallas_call(
        _loss_body,
        out_shape=jax.ShapeDtypeStruct((1, 1), jnp.float32),
    )(lse_g.reshape(128, 128), tgt_acc)
    return out[0, 0]


def kernel(table, idx, targets):
    idx_flat = idx.reshape(ROWS).astype(jnp.int32)
    t_flat = targets.reshape(ROWS).astype(jnp.int32)
    idx2 = jnp.pad(idx_flat.reshape(ROWS // CH, CH), ((0, 0), (0, 16 - CH)))
    t2 = jnp.pad(t_flat.reshape(ROWS // CH, CH), ((0, 0), (0, 16 - CH)))
    lse = _row_lse(table)
    logits2, tgt_acc = _sc_row_gather(table, idx2, t2)
    lse_g = _sc_lse_gather(lse.reshape(VOCAB), idx_flat, tgt_acc)
    loss = _loss(lse_g, tgt_acc)
    return (logits2, loss)


# speedup vs baseline: 1.0536x; 1.0536x over previous
import functools

import jax
import jax.numpy as jnp
from jax import lax
from jax.experimental import pallas as pl
from jax.experimental.pallas import tpu as pltpu
from jax.experimental.pallas import tpu_sc as plsc

VOCAB = 4096
DIM = 4096
ROWS = 16384
NW = 32
RPW = ROWS // NW
CH = 8
NBUF = 3
NCH = RPW // CH
ECH = 128
NECH = RPW // ECH
LSE_BLOCK = 512


def _lse_body(t_ref, o_ref):
    x = t_ref[...]
    m = jnp.max(x, axis=1, keepdims=True)
    s = jnp.sum(jnp.exp(x - m), axis=1, keepdims=True)
    o_ref[...] = m + jnp.log(s)


def _row_lse(table):
    return pl.pallas_call(
        _lse_body,
        grid=(VOCAB // LSE_BLOCK,),
        in_specs=[pl.BlockSpec((LSE_BLOCK, DIM), lambda i: (i, 0))],
        out_specs=pl.BlockSpec((LSE_BLOCK, 1), lambda i: (i, 0)),
        out_shape=jax.ShapeDtypeStruct((VOCAB, 1), jnp.float32),
    )(table)


def _sc_row_gather(table, idx_flat, t_flat):
    mesh = plsc.VectorSubcoreMesh(core_axis_name="c", subcore_axis_name="s")

    @functools.partial(
        pl.kernel,
        out_type=(jax.ShapeDtypeStruct((ROWS, DIM), jnp.float32),
                  jax.ShapeDtypeStruct((NW, 16), jnp.float32)),
        mesh=mesh,
        scratch_types=[
            pltpu.VMEM((RPW,), jnp.int32),
            pltpu.VMEM((RPW + 16,), jnp.int32),
            pltpu.VMEM((16,), jnp.float32),
            pltpu.VMEM((CH, DIM), jnp.float32),
            pltpu.VMEM((CH, DIM), jnp.float32),
            pltpu.VMEM((CH, DIM), jnp.float32),
            pltpu.SemaphoreType.DMA,
            pltpu.SemaphoreType.DMA,
            pltpu.SemaphoreType.DMA,
            pltpu.SemaphoreType.DMA,
            pltpu.SemaphoreType.DMA,
            pltpu.SemaphoreType.DMA,
        ],
    )
    def k(table_hbm, idx_hbm, t_hbm, out_hbm, tgt_hbm,
          idx_v, t_v, acc_v, b0, b1, b2,
          g0, g1, g2, o0, o1, o2):
        bufs = (b0, b1, b2)
        gsems = (g0, g1, g2)
        osems = (o0, o1, o2)
        wid = lax.axis_index("s") * 2 + lax.axis_index("c")
        base = wid * RPW
        pltpu.sync_copy(idx_hbm.at[pl.ds(base, RPW)], idx_v)
        pltpu.sync_copy(t_hbm.at[pl.ds(base, RPW)], t_v.at[pl.ds(0, RPW)])
        acc_v[...] = jnp.zeros((16,), jnp.float32)
        lanes = lax.iota(jnp.int32, 16)
        pltpu.async_copy(table_hbm.at[idx_v.at[pl.ds(0, CH)]],
                         bufs[0], gsems[0])
        pltpu.async_copy(table_hbm.at[idx_v.at[pl.ds(CH, CH)]],
                         bufs[1], gsems[1])

        @pl.loop(0, NCH + 2, step=NBUF)
        def _(gbase):
            for j in range(NBUF):
                g = gbase + j
                b = j
                nb = (j + 2) % NBUF

                @pl.when(jnp.logical_and(g >= 1, g <= NCH))
                def _():
                    pltpu.make_async_copy(
                        bufs[nb], out_hbm.at[pl.ds(base, CH)],
                        osems[nb]).wait()

                @pl.when(g + 2 < NCH)
                def _():
                    pltpu.async_copy(
                        table_hbm.at[idx_v.at[pl.ds((g + 2) * CH, CH)]],
                        bufs[nb], gsems[nb])

                @pl.when(g < NCH)
                def _():
                    pltpu.make_async_copy(
                        table_hbm.at[idx_v.at[pl.ds(0, CH)]],
                        bufs[b], gsems[b]).wait()
                    t16 = t_v[pl.ds(g * CH, 16)]
                    for r in range(CH):
                        t_r = t16[r]
                        a_r = pl.multiple_of((t_r // 16) * 16, 16)
                        v16 = bufs[b][r, pl.ds(a_r, 16)]
                        sel = jnp.where(lanes == t_r - a_r, v16,
                                        jnp.zeros((16,), jnp.float32))
                        acc_v[...] = acc_v[...] + sel
                    pltpu.async_copy(bufs[b],
                                     out_hbm.at[pl.ds(base + g * CH, CH)],
                                     osems[b])

        pltpu.sync_copy(acc_v, tgt_hbm.at[wid])

    return k(table, idx_flat, t_flat)


def _sumlse_body(i_ref, lse_ref, o_ref):
    @pl.when(pl.program_id(0) == 0)
    def _():
        o_ref[...] = jnp.zeros((1, 1), jnp.float32)

    ids = i_ref[0]
    cols = jax.lax.broadcasted_iota(jnp.int32, (VOCAB, 1), 0)
    m = (cols == ids).astype(jnp.float32)
    contrib = jnp.sum(m * lse_ref[...], keepdims=True)
    o_ref[...] = o_ref[...] + contrib


SUMB = 1024


def _sum_lse(lse2d, idx3):
    return pl.pallas_call(
        _sumlse_body,
        grid=(ROWS // SUMB,),
        in_specs=[pl.BlockSpec((1, 1, SUMB), lambda i: (i, 0, 0)),
                  pl.BlockSpec((VOCAB, 1), lambda i: (0, 0))],
        out_specs=pl.BlockSpec((1, 1), lambda i: (0, 0)),
        out_shape=jax.ShapeDtypeStruct((1, 1), jnp.float32),
    )(idx3, lse2d)


def _loss_body(a_ref, b_ref, o_ref):
    sb = jnp.sum(b_ref[...], keepdims=True)
    o_ref[...] = (a_ref[...] - sb) * (1.0 / float(ROWS))


def _loss(sum_lse, tgt_acc):
    out = pl.pallas_call(
        _loss_body,
        out_shape=jax.ShapeDtypeStruct((1, 1), jnp.float32),
    )(sum_lse, tgt_acc)
    return out[0, 0]


def kernel(table, idx, targets):
    idx_flat = idx.reshape(ROWS).astype(jnp.int32)
    t_flat = targets.reshape(ROWS).astype(jnp.int32)
    idx3 = idx_flat.reshape(ROWS // SUMB, 1, SUMB)
    lse = _row_lse(table)
    logits2, tgt_acc = _sc_row_gather(table, idx_flat, t_flat)
    sum_lse = _sum_lse(lse, idx3)
    loss = _loss(sum_lse, tgt_acc)
    return (logits2, loss)


# speedup vs baseline: 1.0552x; 1.0015x over previous
import functools

import jax
import jax.numpy as jnp
from jax import lax
from jax.experimental import pallas as pl
from jax.experimental.pallas import tpu as pltpu
from jax.experimental.pallas import tpu_sc as plsc

VOCAB = 4096
DIM = 4096
ROWS = 16384
NW = 32
RPW = ROWS // NW
CH = 8
NBUF = 3
NCH = RPW // CH
LSE_BLOCK = 512


def _lse_body(t_ref, o_ref):
    x = t_ref[...]
    m = jnp.max(x, axis=1, keepdims=True)
    s = jnp.sum(jnp.exp(x - m), axis=1, keepdims=True)
    o_ref[...] = m + jnp.log(s)


def _row_lse(table):
    return pl.pallas_call(
        _lse_body,
        grid=(VOCAB // LSE_BLOCK,),
        in_specs=[pl.BlockSpec((LSE_BLOCK, DIM), lambda i: (i, 0))],
        out_specs=pl.BlockSpec((LSE_BLOCK, 1), lambda i: (i, 0)),
        out_shape=jax.ShapeDtypeStruct((VOCAB, 1), jnp.float32),
    )(table)


def _sc_row_gather(table, idx_flat, t_flat):
    mesh = plsc.VectorSubcoreMesh(core_axis_name="c", subcore_axis_name="s")

    @functools.partial(
        pl.kernel,
        out_type=(jax.ShapeDtypeStruct((ROWS, DIM), jnp.float32),
                  jax.ShapeDtypeStruct((NW, 16), jnp.float32)),
        mesh=mesh,
        scratch_types=[
            pltpu.VMEM((RPW,), jnp.int32),
            pltpu.VMEM((RPW + 16,), jnp.int32),
            pltpu.VMEM((16,), jnp.float32),
            pltpu.VMEM((CH, DIM), jnp.float32),
            pltpu.VMEM((CH, DIM), jnp.float32),
            pltpu.VMEM((CH, DIM), jnp.float32),
            pltpu.SemaphoreType.DMA,
            pltpu.SemaphoreType.DMA,
            pltpu.SemaphoreType.DMA,
            pltpu.SemaphoreType.DMA,
            pltpu.SemaphoreType.DMA,
            pltpu.SemaphoreType.DMA,
        ],
    )
    def k(table_hbm, idx_hbm, t_hbm, out_hbm, tgt_hbm,
          idx_v, t_v, acc_v, b0, b1, b2,
          g0, g1, g2, o0, o1, o2):
        bufs = (b0, b1, b2)
        gsems = (g0, g1, g2)
        osems = (o0, o1, o2)
        wid = lax.axis_index("s") * 2 + lax.axis_index("c")
        base = wid * RPW
        pltpu.sync_copy(idx_hbm.at[pl.ds(base, RPW)], idx_v)
        pltpu.sync_copy(t_hbm.at[pl.ds(base, RPW)], t_v.at[pl.ds(0, RPW)])
        acc_v[...] = jnp.zeros((16,), jnp.float32)
        lanes = lax.iota(jnp.int32, 16)
        pltpu.async_copy(table_hbm.at[idx_v.at[pl.ds(0, CH)]],
                         bufs[0], gsems[0])
        pltpu.async_copy(table_hbm.at[idx_v.at[pl.ds(CH, CH)]],
                         bufs[1], gsems[1])

        @pl.loop(0, NCH + 2, step=NBUF)
        def _(gbase):
            for j in range(NBUF):
                g = gbase + j
                b = j
                nb = (j + 2) % NBUF

                @pl.when(jnp.logical_and(g >= 1, g <= NCH))
                def _():
                    pltpu.make_async_copy(
                        bufs[nb], out_hbm.at[pl.ds(base, CH)],
                        osems[nb]).wait()

                @pl.when(g + 2 < NCH)
                def _():
                    pltpu.async_copy(
                        table_hbm.at[idx_v.at[pl.ds((g + 2) * CH, CH)]],
                        bufs[nb], gsems[nb])

                @pl.when(g < NCH)
                def _():
                    pltpu.make_async_copy(
                        table_hbm.at[idx_v.at[pl.ds(0, CH)]],
                        bufs[b], gsems[b]).wait()
                    t16 = t_v[pl.ds(g * CH, 16)]
                    for r in range(CH):
                        t_r = t16[r]
                        a_r = pl.multiple_of((t_r // 16) * 16, 16)
                        v16 = bufs[b][r, pl.ds(a_r, 16)]
                        sel = jnp.where(lanes == t_r - a_r, v16,
                                        jnp.zeros((16,), jnp.float32))
                        acc_v[...] = acc_v[...] + sel
                    pltpu.async_copy(bufs[b],
                                     out_hbm.at[pl.ds(base + g * CH, CH)],
                                     osems[b])

        pltpu.sync_copy(acc_v, tgt_hbm.at[wid])

    return k(table, idx_flat, t_flat)


def _sumlse_body(i_ref, lse_ref, o_ref):
    @pl.when(pl.program_id(0) == 0)
    def _():
        o_ref[...] = jnp.zeros((1, 1), jnp.float32)

    ids = i_ref[0]
    cols = jax.lax.broadcasted_iota(jnp.int32, (VOCAB, 1), 0)
    m = (cols == ids).astype(jnp.float32)
    contrib = jnp.sum(m * lse_ref[...], keepdims=True)
    o_ref[...] = o_ref[...] + contrib


SUMB = 1024


def _sum_lse(lse2d, idx3):
    return pl.pallas_call(
        _sumlse_body,
        grid=(ROWS // SUMB,),
        in_specs=[pl.BlockSpec((1, 1, SUMB), lambda i: (i, 0, 0)),
                  pl.BlockSpec((VOCAB, 1), lambda i: (0, 0))],
        out_specs=pl.BlockSpec((1, 1), lambda i: (0, 0)),
        out_shape=jax.ShapeDtypeStruct((1, 1), jnp.float32),
    )(idx3, lse2d)


def _loss_body(a_ref, b_ref, o_ref):
    sb = jnp.sum(b_ref[...], keepdims=True)
    o_ref[...] = (a_ref[...] - sb) * (1.0 / float(ROWS))


def _loss(sum_lse, tgt_acc):
    out = pl.pallas_call(
        _loss_body,
        out_shape=jax.ShapeDtypeStruct((1, 1), jnp.float32),
    )(sum_lse, tgt_acc)
    return out[0, 0]


def kernel(table, idx, targets):
    idx_flat = idx.reshape(ROWS).astype(jnp.int32)
    t_flat = targets.reshape(ROWS).astype(jnp.int32)
    idx3 = idx_flat.reshape(ROWS // SUMB, 1, SUMB)
    lse = _row_lse(table)
    logits2, tgt_acc = _sc_row_gather(table, idx_flat, t_flat)
    sum_lse = _sum_lse(lse, idx3)
    loss = _loss(sum_lse, tgt_acc)
    return (logits2, loss)
